# Initial kernel scaffold; baseline (speedup 1.0000x reference)
import jax
import jax.numpy as jnp
from jax.experimental import pallas as pl


def kernel(poss_node, poss_edge, groundTruth, mask, edges, weights):
    raise NotImplementedError("write your pallas kernel here")


# speedup vs baseline: 13.5235x; 13.5235x over previous
import functools

import jax
import jax.numpy as jnp
from jax import lax
from jax.experimental import pallas as pl
from jax.experimental.pallas import tpu as pltpu
from jax.experimental.pallas import tpu_sc as plsc

_SEMI_LAMBDA = 0.001
_EDGE_LAMBDA = 1.0
_K = 80


def _sc_stage(poss_node, poss_edge, gt, mask_i, e0, e1):
    N, C = poss_node.shape
    E = e0.shape[0]
    info = plsc.get_sparse_core_info()
    NC, NS, L = info.num_cores, info.num_subcores, info.num_lanes
    NW = NC * NS
    EPW = E // NW
    assert EPW * NW == E and EPW % _K == 0
    n_chunks = EPW // _K
    mesh = plsc.VectorSubcoreMesh(core_axis_name="c", subcore_axis_name="s")

    @functools.partial(
        pl.kernel,
        mesh=mesh,
        compiler_params=pltpu.CompilerParams(needs_layout_passes=False),
        out_type=[
            jax.ShapeDtypeStruct((E,), jnp.float32),
            jax.ShapeDtypeStruct((NW, L), jnp.float32),
            jax.ShapeDtypeStruct((NW, L), jnp.float32),
        ],
        scratch_types=[
            pltpu.VMEM((N,), jnp.int32),
            pltpu.VMEM((N,), jnp.int32),
            pltpu.VMEM((_K,), jnp.int32),
            pltpu.VMEM((_K,), jnp.int32),
            pltpu.VMEM((_K, C), jnp.float32),
            pltpu.VMEM((_K, C), jnp.float32),
            pltpu.VMEM((_K, C + 1), jnp.float32),
            pltpu.VMEM((_K,), jnp.float32),
            pltpu.VMEM((L,), jnp.float32),
            pltpu.VMEM((L,), jnp.float32),
            pltpu.SemaphoreType.DMA,
            pltpu.SemaphoreType.DMA,
        ],
    )
    def sc_k(pn_hbm, pe_hbm, gt_hbm, msk_hbm, e0_hbm, e1_hbm,
             p_out, semi_out, cnt_out,
             gt_v, msk_v, e0_v, e1_v, ru_v, rv_v, pe_v, p_v, st0, st1,
             sem0, sem1):
        wid = lax.axis_index("s") * NC + lax.axis_index("c")
        base = wid * EPW
        pltpu.sync_copy(gt_hbm, gt_v)
        pltpu.sync_copy(msk_hbm, msk_v)

        def chunk(j, carry):
            semi_acc, cnt_acc = carry
            off = base + j * _K
            pltpu.sync_copy(e0_hbm.at[pl.ds(off, _K)], e0_v)
            pltpu.sync_copy(e1_hbm.at[pl.ds(off, _K)], e1_v)
            cp_u = pltpu.async_copy(pn_hbm.at[e0_v], ru_v, sem0)
            cp_v = pltpu.async_copy(pn_hbm.at[e1_v], rv_v, sem1)
            pltpu.sync_copy(pe_hbm.at[pl.ds(off, _K)], pe_v)
            for i in range(_K // L):
                ev0 = e0_v[pl.ds(i * L, L)]
                ev1 = e1_v[pl.ds(i * L, L)]
                g0 = plsc.load_gather(gt_v, [ev0])
                g1 = plsc.load_gather(gt_v, [ev1])
                m0 = plsc.load_gather(msk_v, [ev0])
                m1 = plsc.load_gather(msk_v, [ev1])
                rloc = lax.iota(jnp.int32, L) + (i * L)
                v0 = plsc.load_gather(pe_v, [rloc, g0])
                v1 = plsc.load_gather(pe_v, [rloc, g1])
                both = m0 * m1
                cnt_acc = cnt_acc + jnp.maximum(m0, m1).astype(jnp.float32)
                p = jnp.where(both == 1, v0 * v1, jnp.float32(1.0))
                p_v[pl.ds(i * L, L)] = p
            cp_u.wait()
            cp_v.wait()

            def row(r, acc):
                for c in range(C // L):
                    a = ru_v[r, pl.ds(c * L, L)]
                    b = rv_v[r, pl.ds(c * L, L)]
                    d = a - b
                    acc = acc + d * d
                return acc

            semi_acc = lax.fori_loop(0, _K, row, semi_acc)
            pltpu.sync_copy(p_v, p_out.at[pl.ds(off, _K)])
            return semi_acc, cnt_acc

        z = jnp.zeros((L,), jnp.float32)
        semi_acc, cnt_acc = lax.fori_loop(0, n_chunks, chunk, (z, z))
        st0[...] = semi_acc
        st1[...] = cnt_acc
        pltpu.sync_copy(st0, semi_out.at[wid])
        pltpu.sync_copy(st1, cnt_out.at[wid])

    return sc_k(poss_node, poss_edge, gt, mask_i, e0, e1)


def _tc_stage(poss_node, gt3, msk3, p3, semi_parts, cnt_parts):
    G, NB, _ = gt3.shape
    _, PB, C = p3.shape
    NWp, L = semi_parts.shape

    def tc_k(pn_ref, gt_ref, msk_ref, p_ref, semi_ref, cnt_ref, out_ref,
             acc_ref):
        i = pl.program_id(0)

        @pl.when(i == 0)
        def _init():
            acc_ref[0] = 0.0
            acc_ref[1] = 0.0
            acc_ref[2] = 0.0

        pn = pn_ref[...]
        gtb = gt_ref[0]
        mb = msk_ref[0]
        cols = lax.broadcasted_iota(jnp.int32, (NB, C), 1)
        chosen = jnp.sum(jnp.where(cols == gtb, pn, 0.0), axis=1,
                         keepdims=True)
        acc_ref[0] += jnp.sum(jnp.log(chosen) * mb)
        acc_ref[1] += jnp.sum(mb)
        acc_ref[2] += jnp.sum(jnp.log(p_ref[0]))

        @pl.when(i == pl.num_programs(0) - 1)
        def _fin():
            semi = 0.5 * jnp.sum(semi_ref[...])
            cnt = jnp.sum(cnt_ref[...])
            nll = -acc_ref[0] / acc_ref[1]
            edge = (-0.5 * _EDGE_LAMBDA) * acc_ref[2] / cnt
            out_ref[...] = jnp.full((1, 1), nll + _SEMI_LAMBDA * semi + edge,
                                    jnp.float32)

    return pl.pallas_call(
        tc_k,
        grid=(G,),
        in_specs=[
            pl.BlockSpec((NB, C), lambda i: (i, 0)),
            pl.BlockSpec((1, NB, 1), lambda i: (i, 0, 0)),
            pl.BlockSpec((1, NB, 1), lambda i: (i, 0, 0)),
            pl.BlockSpec((1, PB, C), lambda i: (i, 0, 0)),
            pl.BlockSpec((NWp, L), lambda i: (0, 0)),
            pl.BlockSpec((NWp, L), lambda i: (0, 0)),
        ],
        out_specs=pl.BlockSpec((1, 1), lambda i: (0, 0)),
        out_shape=jax.ShapeDtypeStruct((1, 1), jnp.float32),
        scratch_shapes=[pltpu.SMEM((4,), jnp.float32)],
    )(poss_node, gt3, msk3, p3, semi_parts, cnt_parts)


def kernel(poss_node, poss_edge, groundTruth, mask, edges, weights):
    N, C = poss_node.shape
    E = poss_edge.shape[0]
    e0 = edges[:, 0]
    e1 = edges[:, 1]
    mask_i = mask.astype(jnp.int32)
    p, semi_parts, cnt_parts = _sc_stage(poss_node, poss_edge, groundTruth,
                                         mask_i, e0, e1)
    G = 10
    gt3 = groundTruth.reshape(G, N // G, 1)
    msk3 = mask.astype(jnp.float32).reshape(G, N // G, 1)
    p3 = p.reshape(G, E // (G * C), C)
    loss = _tc_stage(poss_node, gt3, msk3, p3, semi_parts, cnt_parts)
    return loss[0, 0]


# speedup vs baseline: 19.2543x; 1.4238x over previous
import functools

import jax
import jax.numpy as jnp
from jax import lax
from jax.experimental import pallas as pl
from jax.experimental.pallas import tpu as pltpu
from jax.experimental.pallas import tpu_sc as plsc

_SEMI_LAMBDA = 0.001
_EDGE_LAMBDA = 1.0
_K = 80


def _sc_stage(poss_node, poss_edge, gm, e0, e1):
    N, C = poss_node.shape
    E = e0.shape[0]
    info = plsc.get_sparse_core_info()
    NC, NS, L = info.num_cores, info.num_subcores, info.num_lanes
    NW = NC * NS
    EPW = E // NW
    assert EPW * NW == E and EPW % _K == 0
    n_chunks = EPW // _K
    mesh = plsc.VectorSubcoreMesh(core_axis_name="c", subcore_axis_name="s")

    assert n_chunks % 2 == 1 and n_chunks >= 3
    T = (n_chunks - 1) // 2

    @functools.partial(
        pl.kernel,
        mesh=mesh,
        compiler_params=pltpu.CompilerParams(needs_layout_passes=False),
        out_type=[
            jax.ShapeDtypeStruct((E,), jnp.float32),
            jax.ShapeDtypeStruct((NW, L), jnp.float32),
            jax.ShapeDtypeStruct((NW, L), jnp.float32),
        ],
        scratch_types=[
            pltpu.VMEM((N,), jnp.int32),
            pltpu.VMEM((EPW,), jnp.int32),
            pltpu.VMEM((EPW,), jnp.int32),
            pltpu.VMEM((_K, C), jnp.float32),
            pltpu.VMEM((_K, C), jnp.float32),
            pltpu.VMEM((_K, C), jnp.float32),
            pltpu.VMEM((_K, C), jnp.float32),
            pltpu.VMEM((_K, C + 1), jnp.float32),
            pltpu.VMEM((_K, C + 1), jnp.float32),
            pltpu.VMEM((EPW,), jnp.float32),
            pltpu.VMEM((L,), jnp.float32),
            pltpu.VMEM((L,), jnp.float32),
            pltpu.SemaphoreType.DMA,
            pltpu.SemaphoreType.DMA,
        ],
    )
    def sc_k(pn_hbm, pe_hbm, gm_hbm, e0_hbm, e1_hbm,
             p_out, semi_out, cnt_out,
             gm_v, e0_v, e1_v, ru0, rv0, ru1, rv1, pe0, pe1, p_all,
             st0, st1, sem0, sem1):
        wid = lax.axis_index("s") * NC + lax.axis_index("c")
        base = wid * EPW
        bufs = ((ru0, rv0, pe0, sem0), (ru1, rv1, pe1, sem1))
        pltpu.sync_copy(gm_hbm, gm_v)
        pltpu.sync_copy(e0_hbm.at[pl.ds(base, EPW)], e0_v)
        pltpu.sync_copy(e1_hbm.at[pl.ds(base, EPW)], e1_v)

        def start(c, b):
            ru, rv, pe, sem = bufs[b]
            loc = c * _K
            pltpu.async_copy(pn_hbm.at[e0_v.at[pl.ds(loc, _K)]], ru, sem)
            pltpu.async_copy(pn_hbm.at[e1_v.at[pl.ds(loc, _K)]], rv, sem)
            pltpu.async_copy(pe_hbm.at[pl.ds(base + loc, _K)], pe, sem)

        def wait(b):
            ru, rv, pe, sem = bufs[b]
            pltpu.make_async_copy(pn_hbm.at[e0_v.at[pl.ds(0, _K)]], ru,
                                  sem).wait()
            pltpu.make_async_copy(pn_hbm.at[e1_v.at[pl.ds(0, _K)]], rv,
                                  sem).wait()
            pltpu.make_async_copy(pe_hbm.at[pl.ds(base, _K)], pe, sem).wait()

        def compute(c, b, semi_acc, cnt_acc):
            ru, rv, pe, _ = bufs[b]
            loc = c * _K
            for i in range(_K // L):
                ev0 = e0_v[pl.ds(loc + i * L, L)]
                ev1 = e1_v[pl.ds(loc + i * L, L)]
                pm0 = plsc.load_gather(gm_v, [ev0])
                pm1 = plsc.load_gather(gm_v, [ev1])
                g0 = pm0 & 255
                g1 = pm1 & 255
                m0 = pm0 >> 8
                m1 = pm1 >> 8
                rloc = lax.iota(jnp.int32, L) + (i * L)
                v0 = plsc.load_gather(pe, [rloc, g0])
                v1 = plsc.load_gather(pe, [rloc, g1])
                both = m0 * m1
                cnt_acc = cnt_acc + jnp.maximum(m0, m1).astype(jnp.float32)
                p = jnp.where(both == 1, v0 * v1, jnp.float32(1.0))
                p_all[pl.ds(loc + i * L, L)] = p

            def row(r, acc):
                for cc in range(C // L):
                    a = ru[r, pl.ds(cc * L, L)]
                    bb = rv[r, pl.ds(cc * L, L)]
                    d = a - bb
                    acc = acc + d * d
                return acc

            semi_acc = plsc.parallel_loop(0, _K, unroll=4,
                                          carry=semi_acc)(row)
            return semi_acc, cnt_acc

        z = jnp.zeros((L,), jnp.float32)
        start(0, 0)
        start(1, 1)
        wait(0)
        carry0 = compute(0, 0, z, z)

        def body(t, carry):
            semi_acc, cnt_acc = carry
            start(2 * t + 2, 0)
            wait(1)
            semi_acc, cnt_acc = compute(2 * t + 1, 1, semi_acc, cnt_acc)

            @pl.when(t < T - 1)
            def _():
                start(2 * t + 3, 1)

            wait(0)
            return compute(2 * t + 2, 0, semi_acc, cnt_acc)

        semi_acc, cnt_acc = lax.fori_loop(0, T, body, carry0)
        st0[...] = semi_acc
        st1[...] = cnt_acc
        pltpu.sync_copy(p_all, p_out.at[pl.ds(base, EPW)])
        pltpu.sync_copy(st0, semi_out.at[wid])
        pltpu.sync_copy(st1, cnt_out.at[wid])

    return sc_k(poss_node, poss_edge, gm, e0, e1)


def _tc_stage(poss_node, gt3, msk3, p3, semi_parts, cnt_parts):
    G, NB, _ = gt3.shape
    _, PB, C = p3.shape
    NWp, L = semi_parts.shape

    def tc_k(pn_ref, gt_ref, msk_ref, p_ref, semi_ref, cnt_ref, out_ref,
             acc_ref):
        i = pl.program_id(0)

        @pl.when(i == 0)
        def _init():
            acc_ref[0] = 0.0
            acc_ref[1] = 0.0
            acc_ref[2] = 0.0

        pn = pn_ref[...]
        gtb = gt_ref[0]
        mb = msk_ref[0]
        cols = lax.broadcasted_iota(jnp.int32, (NB, C), 1)
        chosen = jnp.sum(jnp.where(cols == gtb, pn, 0.0), axis=1,
                         keepdims=True)
        acc_ref[0] += jnp.sum(jnp.log(chosen) * mb)
        acc_ref[1] += jnp.sum(mb)
        acc_ref[2] += jnp.sum(jnp.log(p_ref[0]))

        @pl.when(i == pl.num_programs(0) - 1)
        def _fin():
            semi = 0.5 * jnp.sum(semi_ref[...])
            cnt = jnp.sum(cnt_ref[...])
            nll = -acc_ref[0] / acc_ref[1]
            edge = (-0.5 * _EDGE_LAMBDA) * acc_ref[2] / cnt
            out_ref[...] = jnp.full((1, 1), nll + _SEMI_LAMBDA * semi + edge,
                                    jnp.float32)

    return pl.pallas_call(
        tc_k,
        grid=(G,),
        in_specs=[
            pl.BlockSpec((NB, C), lambda i: (i, 0)),
            pl.BlockSpec((1, NB, 1), lambda i: (i, 0, 0)),
            pl.BlockSpec((1, NB, 1), lambda i: (i, 0, 0)),
            pl.BlockSpec((1, PB, C), lambda i: (i, 0, 0)),
            pl.BlockSpec((NWp, L), lambda i: (0, 0)),
            pl.BlockSpec((NWp, L), lambda i: (0, 0)),
        ],
        out_specs=pl.BlockSpec((1, 1), lambda i: (0, 0)),
        out_shape=jax.ShapeDtypeStruct((1, 1), jnp.float32),
        scratch_shapes=[pltpu.SMEM((4,), jnp.float32)],
    )(poss_node, gt3, msk3, p3, semi_parts, cnt_parts)


def kernel(poss_node, poss_edge, groundTruth, mask, edges, weights):
    N, C = poss_node.shape
    E = poss_edge.shape[0]
    e0 = edges[:, 0]
    e1 = edges[:, 1]
    gm = groundTruth | (mask.astype(jnp.int32) << 8)
    p, semi_parts, cnt_parts = _sc_stage(poss_node, poss_edge, gm, e0, e1)
    G = 10
    gt3 = groundTruth.reshape(G, N // G, 1)
    msk3 = mask.astype(jnp.float32).reshape(G, N // G, 1)
    p3 = p.reshape(G, E // (G * C), C)
    loss = _tc_stage(poss_node, gt3, msk3, p3, semi_parts, cnt_parts)
    return loss[0, 0]


# speedup vs baseline: 23.4768x; 1.2193x over previous
import functools

import jax
import jax.numpy as jnp
from jax import lax
from jax.experimental import pallas as pl
from jax.experimental.pallas import tpu as pltpu
from jax.experimental.pallas import tpu_sc as plsc

_SEMI_LAMBDA = 0.001
_EDGE_LAMBDA = 1.0
_K = 80
_LN2 = 0.6931471805599453


def _ln(w):
    bw = plsc.bitcast(w, jnp.int32)
    ex = ((bw >> 23) & 0xFF) - 127
    m = plsc.bitcast((bw & 0x7FFFFF) | 0x3F800000, jnp.float32)
    s = (m - 1.0) / (m + 1.0)
    s2 = s * s
    poly = 1.0 + s2 * (1.0 / 3.0 + s2 * (1.0 / 5.0 + s2 * (1.0 / 7.0
                                                           + s2 * (1.0 / 9.0))))
    return 2.0 * s * poly + ex.astype(jnp.float32) * _LN2


def _sc_info():
    info = plsc.get_sparse_core_info()
    return info.num_cores, info.num_subcores, info.num_lanes


def _sc_semi(poss_node, e0, e1):
    N, C = poss_node.shape
    E = e0.shape[0]
    NC, NS, L = _sc_info()
    NW = NC * NS
    EPW = E // NW
    assert EPW * NW == E and EPW % _K == 0
    n_chunks = EPW // _K
    assert n_chunks % 2 == 1 and n_chunks >= 3
    T = (n_chunks - 1) // 2
    mesh = plsc.VectorSubcoreMesh(core_axis_name="c", subcore_axis_name="s")

    @functools.partial(
        pl.kernel,
        mesh=mesh,
        compiler_params=pltpu.CompilerParams(needs_layout_passes=False),
        out_type=jax.ShapeDtypeStruct((NW, L), jnp.float32),
        scratch_types=[
            pltpu.VMEM((EPW,), jnp.int32),
            pltpu.VMEM((EPW,), jnp.int32),
            pltpu.VMEM((_K, C), jnp.float32),
            pltpu.VMEM((_K, C), jnp.float32),
            pltpu.VMEM((_K, C), jnp.float32),
            pltpu.VMEM((_K, C), jnp.float32),
            pltpu.VMEM((L,), jnp.float32),
            pltpu.SemaphoreType.DMA,
            pltpu.SemaphoreType.DMA,
        ],
    )
    def sc_k(pn_hbm, e0_hbm, e1_hbm, semi_out,
             e0_v, e1_v, ru0, rv0, ru1, rv1, st0, sem0, sem1):
        wid = lax.axis_index("s") * NC + lax.axis_index("c")
        base = wid * EPW
        bufs = ((ru0, rv0, sem0), (ru1, rv1, sem1))
        pltpu.sync_copy(e0_hbm.at[pl.ds(base, EPW)], e0_v)
        pltpu.sync_copy(e1_hbm.at[pl.ds(base, EPW)], e1_v)

        def start(c, b):
            ru, rv, sem = bufs[b]
            loc = c * _K
            pltpu.async_copy(pn_hbm.at[e0_v.at[pl.ds(loc, _K)]], ru, sem)
            pltpu.async_copy(pn_hbm.at[e1_v.at[pl.ds(loc, _K)]], rv, sem)

        def wait(b):
            ru, rv, sem = bufs[b]
            pltpu.make_async_copy(pn_hbm.at[e0_v.at[pl.ds(0, _K)]], ru,
                                  sem).wait()
            pltpu.make_async_copy(pn_hbm.at[e1_v.at[pl.ds(0, _K)]], rv,
                                  sem).wait()

        def compute(b, semi_acc):
            ru, rv, _ = bufs[b]

            def row(r, acc):
                for cc in range(C // L):
                    a = ru[r, pl.ds(cc * L, L)]
                    bb = rv[r, pl.ds(cc * L, L)]
                    d = a - bb
                    acc = acc + d * d
                return acc

            return plsc.parallel_loop(0, _K, unroll=4, carry=semi_acc)(row)

        z = jnp.zeros((L,), jnp.float32)
        start(0, 0)
        start(1, 1)
        wait(0)
        carry0 = compute(0, z)

        def body(t, semi_acc):
            start(2 * t + 2, 0)
            wait(1)
            semi_acc = compute(1, semi_acc)

            @pl.when(t < T - 1)
            def _():
                start(2 * t + 3, 1)

            wait(0)
            return compute(0, semi_acc)

        semi_acc = lax.fori_loop(0, T, body, carry0)
        st0[...] = semi_acc
        pltpu.sync_copy(st0, semi_out.at[wid])

    return sc_k(poss_node, e0, e1)


def _sc_edge(pe_flat, gm, e0, e1):
    N = gm.shape[0]
    E = e0.shape[0]
    C = 128
    NC, NS, L = _sc_info()
    NW = NC * NS
    EPW = E // NW
    n_chunks = EPW // _K
    T = (n_chunks - 1) // 2
    mesh = plsc.VectorSubcoreMesh(core_axis_name="c", subcore_axis_name="s")

    @functools.partial(
        pl.kernel,
        mesh=mesh,
        compiler_params=pltpu.CompilerParams(needs_layout_passes=False),
        out_type=[
            jax.ShapeDtypeStruct((NW, L), jnp.float32),
            jax.ShapeDtypeStruct((NW, L), jnp.float32),
        ],
        scratch_types=[
            pltpu.VMEM((N,), jnp.int32),
            pltpu.VMEM((EPW,), jnp.int32),
            pltpu.VMEM((EPW,), jnp.int32),
            pltpu.VMEM((_K,), jnp.int32),
            pltpu.VMEM((_K,), jnp.int32),
            pltpu.VMEM((_K,), jnp.int32),
            pltpu.VMEM((_K,), jnp.int32),
            pltpu.VMEM((_K,), jnp.float32),
            pltpu.VMEM((_K,), jnp.float32),
            pltpu.VMEM((_K,), jnp.float32),
            pltpu.VMEM((_K,), jnp.float32),
            pltpu.VMEM((_K,), jnp.float32),
            pltpu.VMEM((_K,), jnp.float32),
            pltpu.VMEM((_K,), jnp.float32),
            pltpu.VMEM((_K,), jnp.float32),
            pltpu.VMEM((L,), jnp.float32),
            pltpu.VMEM((L,), jnp.float32),
            pltpu.SemaphoreType.DMA,
            pltpu.SemaphoreType.DMA,
        ],
    )
    def sc_k(pe_hbm, gm_hbm, e0_hbm, e1_hbm,
             edge_out, cnt_out,
             gm_v, e0_v, e1_v, iu0, iv0, iu1, iv1, vu0, vv0, vu1, vv1,
             bo0, bo1, lb0, lb1, st0, st1, sem0, sem1):
        wid = lax.axis_index("s") * NC + lax.axis_index("c")
        base = wid * EPW
        bufs = ((iu0, iv0, vu0, vv0, bo0, lb0, sem0),
                (iu1, iv1, vu1, vv1, bo1, lb1, sem1))
        pltpu.sync_copy(gm_hbm, gm_v)
        pltpu.sync_copy(e0_hbm.at[pl.ds(base, EPW)], e0_v)
        pltpu.sync_copy(e1_hbm.at[pl.ds(base, EPW)], e1_v)

        def build(c, b):
            iu, iv, _, _, bo, lb, _ = bufs[b]
            loc = c * _K
            for i in range(_K // L):
                ev0 = e0_v[pl.ds(loc + i * L, L)]
                ev1 = e1_v[pl.ds(loc + i * L, L)]
                pm0 = plsc.load_gather(gm_v, [ev0])
                pm1 = plsc.load_gather(gm_v, [ev1])
                g0 = pm0 & 255
                g1 = pm1 & 255
                m0 = pm0 >> 8
                m1 = pm1 >> 8
                eg = lax.iota(jnp.int32, L) + (base + loc + i * L)
                iu[pl.ds(i * L, L)] = eg * C + g0
                iv[pl.ds(i * L, L)] = eg * C + g1
                bo[pl.ds(i * L, L)] = (m0 * m1).astype(jnp.float32)
                lb[pl.ds(i * L, L)] = jnp.maximum(m0, m1).astype(jnp.float32)

        def start(b):
            iu, iv, vu, vv, _, _, sem = bufs[b]
            pltpu.async_copy(pe_hbm.at[iu], vu, sem)
            pltpu.async_copy(pe_hbm.at[iv], vv, sem)

        def wait(b):
            iu, iv, vu, vv, _, _, sem = bufs[b]
            pltpu.make_async_copy(pe_hbm.at[iu], vu, sem).wait()
            pltpu.make_async_copy(pe_hbm.at[iv], vv, sem).wait()

        def lnacc(b, edge_acc, cnt_acc):
            _, _, vu, vv, bo, lb, _ = bufs[b]
            for i in range(_K // L):
                v0 = vu[pl.ds(i * L, L)]
                v1 = vv[pl.ds(i * L, L)]
                bf = bo[pl.ds(i * L, L)]
                edge_acc = edge_acc + bf * _ln(v0 * v1)
                cnt_acc = cnt_acc + lb[pl.ds(i * L, L)]
            return edge_acc, cnt_acc

        z = jnp.zeros((L,), jnp.float32)
        build(0, 0)
        start(0)
        build(1, 1)
        start(1)
        wait(0)
        carry0 = lnacc(0, z, z)

        def body(t, carry):
            edge_acc, cnt_acc = carry
            build(2 * t + 2, 0)
            start(0)
            wait(1)
            edge_acc, cnt_acc = lnacc(1, edge_acc, cnt_acc)

            @pl.when(t < T - 1)
            def _():
                build(2 * t + 3, 1)
                start(1)

            wait(0)
            return lnacc(0, edge_acc, cnt_acc)

        edge_acc, cnt_acc = lax.fori_loop(0, T, body, carry0)
        st0[...] = edge_acc
        pltpu.sync_copy(st0, edge_out.at[wid])
        st1[...] = cnt_acc
        pltpu.sync_copy(st1, cnt_out.at[wid])

    return sc_k(pe_flat, gm, e0, e1)


def _tc_node(poss_node, gt3, msk3):
    G, NB, _ = gt3.shape
    C = poss_node.shape[1]

    def k(pn_ref, gt_ref, msk_ref, out_ref):
        i = pl.program_id(0)
        pn = pn_ref[...]
        gtb = gt_ref[0]
        mb = msk_ref[0]
        cols = lax.broadcasted_iota(jnp.int32, (NB, C), 1)
        chosen = jnp.sum(jnp.where(cols == gtb, pn, 0.0), axis=1,
                         keepdims=True)
        num = jnp.sum(jnp.log(chosen) * mb)
        den = jnp.sum(mb)
        lane = lax.broadcasted_iota(jnp.int32, (1, C), 1)
        row = jnp.where(lane == 0, num, jnp.where(lane == 1, den, 0.0))

        @pl.when(i == 0)
        def _first():
            out_ref[...] = row

        @pl.when(i > 0)
        def _rest():
            out_ref[...] += row

    return pl.pallas_call(
        k,
        grid=(G,),
        in_specs=[
            pl.BlockSpec((NB, C), lambda i: (i, 0)),
            pl.BlockSpec((1, NB, 1), lambda i: (i, 0, 0)),
            pl.BlockSpec((1, NB, 1), lambda i: (i, 0, 0)),
        ],
        out_specs=pl.BlockSpec((1, C), lambda i: (0, 0)),
        out_shape=jax.ShapeDtypeStruct((1, C), jnp.float32),
    )(poss_node, gt3, msk3)


def _tc_combine(node_acc, semi_parts, cnt_parts, edge_parts):
    C = node_acc.shape[1]

    def k(nd_ref, se_ref, ct_ref, ed_ref, out_ref):
        nd = nd_ref[...]
        lane = lax.broadcasted_iota(jnp.int32, (1, C), 1)
        num = jnp.sum(jnp.where(lane == 0, nd, 0.0))
        den = jnp.sum(jnp.where(lane == 1, nd, 0.0))
        semi = 0.5 * jnp.sum(se_ref[...])
        cnt = jnp.sum(ct_ref[...])
        esum = jnp.sum(ed_ref[...])
        nll = -num / den
        edge = (-0.5 * _EDGE_LAMBDA) * esum / cnt
        out_ref[...] = jnp.full((1, 1), nll + _SEMI_LAMBDA * semi + edge,
                                jnp.float32)

    return pl.pallas_call(
        k,
        out_shape=jax.ShapeDtypeStruct((1, 1), jnp.float32),
    )(node_acc, semi_parts, cnt_parts, edge_parts)


def kernel(poss_node, poss_edge, groundTruth, mask, edges, weights):
    N, C = poss_node.shape
    E = poss_edge.shape[0]
    e0 = edges[:, 0]
    e1 = edges[:, 1]
    gm = groundTruth | (mask.astype(jnp.int32) << 8)
    pe_flat = poss_edge[:, :C].reshape(-1)
    semi_parts = _sc_semi(poss_node, e0, e1)
    edge_parts, cnt_parts = _sc_edge(pe_flat, gm, e0, e1)
    G = 10
    gt3 = groundTruth.reshape(G, N // G, 1)
    msk3 = mask.astype(jnp.float32).reshape(G, N // G, 1)
    node_acc = _tc_node(poss_node, gt3, msk3)
    loss = _tc_combine(node_acc, semi_parts, cnt_parts, edge_parts)
    return loss[0, 0]
